# baseline (speedup 1.0000x reference)
import functools

import jax
import jax.numpy as jnp
from jax import lax
from jax.experimental import pallas as pl
from jax.experimental.pallas import tpu as pltpu
from jax.experimental.pallas import tpu_sc as plsc

B = 10000
D = 128
HD = D // 2
E = 160000
NREL = 2
NC = 2
NS = 16
NW = NC * NS
LANES = 16

CH = 125
EDGES_PER_TILE = E // NS
NCH = EDGES_PER_TILE // CH

B_PAD = 10240
SEG_ROWS = B_PAD // NS
ZCH = 128
NZ = SEG_ROWS // ZCH
SELF_TILES = 25
SELF_PER_TILE = B // SELF_TILES
SCH = 80
NSCH = SELF_PER_TILE // SCH

_mesh = plsc.VectorSubcoreMesh(core_axis_name="c", subcore_axis_name="s")


@functools.partial(
    pl.kernel,
    out_type=(
        jax.ShapeDtypeStruct((B, D), jnp.float32),
        jax.ShapeDtypeStruct((NREL, B_PAD, D), jnp.float32),
        jax.ShapeDtypeStruct((NREL, B_PAD, LANES), jnp.float32),
    ),
    mesh=_mesh,
    compiler_params=pltpu.CompilerParams(use_tc_tiling_on_sc=False),
    scratch_types=[
        pltpu.VMEM((NCH, CH), jnp.int32),
        pltpu.VMEM((NCH, CH), jnp.int32),
        pltpu.VMEM((ZCH, HD), jnp.float32),
        pltpu.VMEM((ZCH, HD), jnp.float32),
        pltpu.VMEM((ZCH, HD), jnp.float32),
        pltpu.VMEM((ZCH, HD), jnp.float32),
        pltpu.VMEM((CH, LANES), jnp.float32),
        pltpu.VMEM((NSCH, SCH), jnp.int32),
        pltpu.VMEM((SCH, D), jnp.float32),
        pltpu.VMEM((ZCH, LANES), jnp.float32),
        pltpu.VMEM_SHARED((B_PAD, HD), jnp.float32),
        pltpu.VMEM_SHARED((B_PAD, LANES), jnp.float32),
        pltpu.SemaphoreType.DMA,
        pltpu.SemaphoreType.DMA,
        pltpu.SemaphoreType.DMA,
    ],
)
def _aggregate(feat_hbm, featl_hbm, featr_hbm, nodes_hbm, e0_hbm, e1_hbm,
               self_out, acc_out, cnt_out,
               src_v, dst_v, rows_a, rows_b, rows_c, rows_d,
               ones_v, nidx_v, srows_v,
               zcnt_v, acc_sh, cnt_sh, sem_g, sem_s, sem_c):
    c = lax.axis_index("c")
    s = lax.axis_index("s")
    wid = c * NS + s

    zeros16 = jnp.zeros((LANES,), jnp.float32)
    ones16 = jnp.ones((LANES,), jnp.float32)

    def _zero_rows_a():
        def _zr(i, carry):
            for j in range(HD // LANES):
                rows_a[i, pl.ds(j * LANES, LANES)] = zeros16
            return carry

        lax.fori_loop(0, ZCH, _zr, 0)

    _zero_rows_a()

    def _init_row(i, carry):
        zcnt_v[i, :] = zeros16
        return carry

    lax.fori_loop(0, ZCH, _init_row, 0)

    def _init_ones(i, carry):
        ones_v[i, :] = ones16
        return carry

    lax.fori_loop(0, CH, _init_ones, 0)

    def _zero_acc():
        for j in range(NZ):
            pltpu.sync_copy(
                rows_a, acc_sh.at[pl.ds(s * SEG_ROWS + j * ZCH, ZCH)])

    _zero_acc()
    for j in range(NZ):
        pltpu.sync_copy(zcnt_v, cnt_sh.at[pl.ds(s * SEG_ROWS + j * ZCH, ZCH)])
    plsc.subcore_barrier()

    @pl.when(c == 0)
    def _():
        pltpu.sync_copy(e0_hbm.at[1, s], src_v)
        pltpu.sync_copy(e0_hbm.at[0, s], dst_v)

    @pl.when(c == 1)
    def _():
        pltpu.sync_copy(e1_hbm.at[1, s], src_v)
        pltpu.sync_copy(e1_hbm.at[0, s], dst_v)

    bufs = (rows_a, rows_b, rows_c, rows_d)
    NB = 4
    DEPTH = 2

    def _edge_phase(table_hbm, cnt_lo, cnt_hi):
        def _gather_src(j):
            return table_hbm.at[src_v.at[j]]

        def _rows(buf):
            return buf.at[pl.ds(0, CH)]

        for t in range(DEPTH):
            pltpu.async_copy(_gather_src(t), _rows(bufs[t]), sem_g)

        def _hex(i, carry):
            for t in range(NB):
                j = NB * i + t
                buf = bufs[t]
                bufd = bufs[(t + DEPTH) % NB]
                pltpu.make_async_copy(
                    _gather_src(j), _rows(buf), sem_g).wait()
                pltpu.async_copy(
                    _rows(buf), acc_sh.at[dst_v.at[j]], sem_s, add=True)

                @pl.when(jnp.logical_and(cnt_lo <= j, j < cnt_hi))
                def _():
                    pltpu.async_copy(
                        ones_v, cnt_sh.at[dst_v.at[j]], sem_c, add=True)

                @pl.when(j >= DEPTH)
                def _():
                    pltpu.make_async_copy(
                        _rows(bufd), acc_sh.at[dst_v.at[j - DEPTH]],
                        sem_s).wait()

                @pl.when(j + DEPTH < NCH)
                def _():
                    pltpu.async_copy(
                        _gather_src(j + DEPTH), _rows(bufd), sem_g)
            return carry

        lax.fori_loop(0, NCH // NB, _hex, 0)

        for j in range(NCH - NCH % NB, NCH):
            buf = bufs[j % NB]
            bufd = bufs[(j + DEPTH) % NB]
            pltpu.make_async_copy(_gather_src(j), _rows(buf), sem_g).wait()
            pltpu.async_copy(
                _rows(buf), acc_sh.at[dst_v.at[j]], sem_s, add=True)

            @pl.when(jnp.logical_and(cnt_lo <= j, j < cnt_hi))
            def _():
                pltpu.async_copy(
                    ones_v, cnt_sh.at[dst_v.at[j]], sem_c, add=True)

            pltpu.make_async_copy(
                _rows(bufd), acc_sh.at[dst_v.at[j - DEPTH]], sem_s).wait()
        for j in range(NCH - DEPTH, NCH):
            pltpu.make_async_copy(
                _rows(bufs[j % NB]), acc_sh.at[dst_v.at[j]], sem_s).wait()

        def _drain(j, carry):
            @pl.when(jnp.logical_and(cnt_lo <= j, j < cnt_hi))
            def _():
                pltpu.make_async_copy(
                    ones_v, cnt_sh.at[dst_v.at[j]], sem_c).wait()
            return carry

        lax.fori_loop(0, NCH, _drain, 0)

    _edge_phase(featl_hbm, 0, NCH // 2)

    plsc.subcore_barrier()
    for j in range(NZ):
        base = s * SEG_ROWS + j * ZCH
        pltpu.sync_copy(acc_sh.at[pl.ds(base, ZCH)],
                        acc_out.at[c, pl.ds(base, ZCH), pl.ds(0, HD)])
    _zero_rows_a()
    _zero_acc()
    plsc.subcore_barrier()

    _edge_phase(featr_hbm, NCH // 2, NCH)

    @pl.when(wid < SELF_TILES)
    def _():
        pltpu.sync_copy(nodes_hbm.at[wid], nidx_v)
        for t in range(NSCH):
            pltpu.async_copy(feat_hbm.at[nidx_v.at[t]], srows_v, sem_g).wait()
            pltpu.sync_copy(
                srows_v,
                self_out.at[pl.ds(wid * SELF_PER_TILE + t * SCH, SCH)])

    plsc.subcore_barrier()
    for j in range(NZ):
        base = s * SEG_ROWS + j * ZCH
        pltpu.sync_copy(acc_sh.at[pl.ds(base, ZCH)],
                        acc_out.at[c, pl.ds(base, ZCH), pl.ds(HD, HD)])
        pltpu.sync_copy(cnt_sh.at[pl.ds(base, ZCH)],
                        cnt_out.at[c, pl.ds(base, ZCH)])


BLK = 2000


def _tdot(x, w):
    return lax.dot_general(x, w, (((1,), (1,)), ((), ())),
                           preferred_element_type=jnp.float32)


def _mlp_body(self_ref, a0_ref, a1_ref, c0_ref, c1_ref,
              w1_ref, b1_ref, w2_ref, b2_ref, out_ref):
    inv0 = 1.0 / jnp.maximum(c0_ref[:, 0:1], 1.0)
    inv1 = 1.0 / jnp.maximum(c1_ref[:, 0:1], 1.0)
    h = _tdot(self_ref[:], w1_ref[:, :D])
    h = h + _tdot(a0_ref[:] * inv0, w1_ref[:, D:2 * D])
    h = h + _tdot(a1_ref[:] * inv1, w1_ref[:, 2 * D:])
    h = jnp.tanh(h + b1_ref[:])
    out_ref[:] = _tdot(h, w2_ref[:]) + b2_ref[:]


def _mlp(self_feats, a0, a1, c0, c1, w1, b1, w2, b2):
    row_spec = pl.BlockSpec((BLK, D), lambda i: (i, 0))
    cnt_spec = pl.BlockSpec((BLK, LANES), lambda i: (i, 0))

    def full(shape):
        return pl.BlockSpec(shape, lambda *_: (0,) * len(shape))

    return pl.pallas_call(
        _mlp_body,
        grid=(B // BLK,),
        in_specs=[row_spec, row_spec, row_spec, cnt_spec, cnt_spec,
                  full((D, 3 * D)), full((D,)), full((D, D)), full((D,))],
        out_specs=row_spec,
        out_shape=jax.ShapeDtypeStruct((B, D), jnp.float32),
    )(self_feats, a0, a1, c0, c1, w1, b1, w2, b2)


def kernel(nodes, edge_index_0, edge_index_1, feat_table, W1, b1, W2, b2):
    nodes_r = nodes.astype(jnp.int32).reshape(SELF_TILES, NSCH, SCH)
    e0 = edge_index_0.astype(jnp.int32).reshape(2, NS, NCH, CH)
    e1 = edge_index_1.astype(jnp.int32).reshape(2, NS, NCH, CH)
    featl = feat_table[:, :HD]
    featr = feat_table[:, HD:]

    self_rows, acc, cnt = _aggregate(
        feat_table, featl, featr, nodes_r, e0, e1)

    return _mlp(self_rows, acc[0], acc[1], cnt[0], cnt[1], W1, b1, W2, b2)


# speedup vs baseline: 1.0417x; 1.0417x over previous
import functools

import jax
import jax.numpy as jnp
from jax import lax
from jax.experimental import pallas as pl
from jax.experimental.pallas import tpu as pltpu
from jax.experimental.pallas import tpu_sc as plsc

B = 10000
D = 128
HD = D // 2
CW = 8
E = 160000
NREL = 2
NC = 2
NS = 16
NW = NC * NS
LANES = 16

CH = 125
EDGES_PER_TILE = E // NS
NCH = EDGES_PER_TILE // CH

B_PAD = 10240
SEG_ROWS = B_PAD // NS
ZCH = 128
NZ = SEG_ROWS // ZCH
SELF_TILES = 25
SELF_PER_TILE = B // SELF_TILES
SCH = 80
NSCH = SELF_PER_TILE // SCH

_mesh = plsc.VectorSubcoreMesh(core_axis_name="c", subcore_axis_name="s")


@functools.partial(
    pl.kernel,
    out_type=(
        jax.ShapeDtypeStruct((B, D), jnp.float32),
        jax.ShapeDtypeStruct((NREL, B_PAD, D), jnp.float32),
        jax.ShapeDtypeStruct((NREL, B_PAD, CW), jnp.float32),
    ),
    mesh=_mesh,
    compiler_params=pltpu.CompilerParams(use_tc_tiling_on_sc=False),
    scratch_types=[
        pltpu.VMEM((NCH, CH), jnp.int32),
        pltpu.VMEM((NCH, CH), jnp.int32),
        pltpu.VMEM((ZCH, HD), jnp.float32),
        pltpu.VMEM((ZCH, HD), jnp.float32),
        pltpu.VMEM((ZCH, HD), jnp.float32),
        pltpu.VMEM((ZCH, HD), jnp.float32),
        pltpu.VMEM((ZCH, HD), jnp.float32),
        pltpu.VMEM((ZCH, HD), jnp.float32),
        pltpu.VMEM((CH, CW), jnp.float32),
        pltpu.VMEM((NSCH, SCH), jnp.int32),
        pltpu.VMEM((SCH, D), jnp.float32),
        pltpu.VMEM((ZCH, LANES), jnp.float32),
        pltpu.VMEM_SHARED((B_PAD, HD), jnp.float32),
        pltpu.VMEM_SHARED((B_PAD, CW), jnp.float32),
        pltpu.SemaphoreType.DMA,
        pltpu.SemaphoreType.DMA,
        pltpu.SemaphoreType.DMA,
    ],
)
def _aggregate(feat_hbm, featl_hbm, featr_hbm, nodes_hbm, e0_hbm, e1_hbm,
               ones_hbm,
               self_out, acc_out, cnt_out,
               src_v, dst_v, rows_a, rows_b, rows_e, rows_f, rows_c, rows_d,
               ones_v, nidx_v, srows_v,
               zcnt_v, acc_sh, cnt_sh, sem_g, sem_s, sem_c):
    c = lax.axis_index("c")
    s = lax.axis_index("s")
    wid = c * NS + s

    zeros16 = jnp.zeros((LANES,), jnp.float32)
    ones16 = jnp.ones((LANES,), jnp.float32)

    def _zero_rows_a():
        def _zr(i, carry):
            for j in range(HD // LANES):
                rows_a[i, pl.ds(j * LANES, LANES)] = zeros16
            return carry

        lax.fori_loop(0, ZCH, _zr, 0)

    _zero_rows_a()

    def _init_row(i, carry):
        zcnt_v[i, :] = zeros16
        return carry

    lax.fori_loop(0, ZCH, _init_row, 0)

    pltpu.sync_copy(ones_hbm, ones_v)

    def _zero_acc():
        for j in range(NZ):
            pltpu.sync_copy(
                rows_a, acc_sh.at[pl.ds(s * SEG_ROWS + j * ZCH, ZCH)])

    _zero_acc()
    for j in range(NZ):
        pltpu.sync_copy(zcnt_v.at[pl.ds(0, ZCH), pl.ds(0, CW)],
                        cnt_sh.at[pl.ds(s * SEG_ROWS + j * ZCH, ZCH)])
    plsc.subcore_barrier()

    @pl.when(c == 0)
    def _():
        pltpu.sync_copy(e0_hbm.at[1, s], src_v)
        pltpu.sync_copy(e0_hbm.at[0, s], dst_v)

    @pl.when(c == 1)
    def _():
        pltpu.sync_copy(e1_hbm.at[1, s], src_v)
        pltpu.sync_copy(e1_hbm.at[0, s], dst_v)

    bufs = (rows_a, rows_b, rows_e, rows_f, rows_c, rows_d)
    NB = 6
    DEPTH = 3
    ones_pay = ones_v

    def _edge_phase(table_hbm, cnt_lo, cnt_hi):
        def _gather_src(j):
            return table_hbm.at[src_v.at[j]]

        def _rows(buf):
            return buf.at[pl.ds(0, CH)]

        for t in range(DEPTH):
            pltpu.async_copy(_gather_src(t), _rows(bufs[t]), sem_g)

        def _hex(i, carry):
            for t in range(NB):
                j = NB * i + t
                buf = bufs[t]
                bufd = bufs[(t + DEPTH) % NB]
                pltpu.make_async_copy(
                    _gather_src(j), _rows(buf), sem_g).wait()
                pltpu.async_copy(
                    _rows(buf), acc_sh.at[dst_v.at[j]], sem_s, add=True)

                @pl.when(jnp.logical_and(cnt_lo <= j, j < cnt_hi))
                def _():
                    pltpu.async_copy(
                        ones_pay, cnt_sh.at[dst_v.at[j]], sem_c, add=True)

                @pl.when(j >= DEPTH)
                def _():
                    pltpu.make_async_copy(
                        _rows(bufd), acc_sh.at[dst_v.at[j - DEPTH]],
                        sem_s).wait()

                @pl.when(j + DEPTH < NCH)
                def _():
                    pltpu.async_copy(
                        _gather_src(j + DEPTH), _rows(bufd), sem_g)
            return carry

        lax.fori_loop(0, NCH // NB, _hex, 0)

        for j in range(NCH - NCH % NB, NCH):
            buf = bufs[j % NB]
            bufd = bufs[(j + DEPTH) % NB]
            pltpu.make_async_copy(_gather_src(j), _rows(buf), sem_g).wait()
            pltpu.async_copy(
                _rows(buf), acc_sh.at[dst_v.at[j]], sem_s, add=True)

            @pl.when(jnp.logical_and(cnt_lo <= j, j < cnt_hi))
            def _():
                pltpu.async_copy(
                    ones_pay, cnt_sh.at[dst_v.at[j]], sem_c, add=True)

            pltpu.make_async_copy(
                _rows(bufd), acc_sh.at[dst_v.at[j - DEPTH]], sem_s).wait()
        for j in range(NCH - DEPTH, NCH):
            pltpu.make_async_copy(
                _rows(bufs[j % NB]), acc_sh.at[dst_v.at[j]], sem_s).wait()

        def _drain(j, carry):
            @pl.when(jnp.logical_and(cnt_lo <= j, j < cnt_hi))
            def _():
                pltpu.make_async_copy(
                    ones_pay, cnt_sh.at[dst_v.at[j]], sem_c).wait()
            return carry

        lax.fori_loop(0, NCH, _drain, 0)

    _edge_phase(featl_hbm, 0, NCH // 2)

    plsc.subcore_barrier()
    for j in range(NZ):
        base = s * SEG_ROWS + j * ZCH
        pltpu.sync_copy(acc_sh.at[pl.ds(base, ZCH)],
                        acc_out.at[c, pl.ds(base, ZCH), pl.ds(0, HD)])
    _zero_rows_a()
    _zero_acc()
    plsc.subcore_barrier()

    _edge_phase(featr_hbm, NCH // 2, NCH)

    @pl.when(wid < SELF_TILES)
    def _():
        pltpu.sync_copy(nodes_hbm.at[wid], nidx_v)
        for t in range(NSCH):
            pltpu.async_copy(feat_hbm.at[nidx_v.at[t]], srows_v, sem_g).wait()
            pltpu.sync_copy(
                srows_v,
                self_out.at[pl.ds(wid * SELF_PER_TILE + t * SCH, SCH)])

    plsc.subcore_barrier()
    for j in range(NZ):
        base = s * SEG_ROWS + j * ZCH
        pltpu.sync_copy(acc_sh.at[pl.ds(base, ZCH)],
                        acc_out.at[c, pl.ds(base, ZCH), pl.ds(HD, HD)])
        pltpu.sync_copy(cnt_sh.at[pl.ds(base, ZCH)],
                        cnt_out.at[c, pl.ds(base, ZCH)])


BLK = 2000


def _tdot(x, w):
    return lax.dot_general(x, w, (((1,), (1,)), ((), ())),
                           preferred_element_type=jnp.float32)


def _mlp_body(self_ref, a0_ref, a1_ref, c0_ref, c1_ref,
              w1_ref, b1_ref, w2_ref, b2_ref, out_ref):
    inv0 = 1.0 / jnp.maximum(c0_ref[:, 0:1], 1.0)
    inv1 = 1.0 / jnp.maximum(c1_ref[:, 0:1], 1.0)
    h = _tdot(self_ref[:], w1_ref[:, :D])
    h = h + _tdot(a0_ref[:] * inv0, w1_ref[:, D:2 * D])
    h = h + _tdot(a1_ref[:] * inv1, w1_ref[:, 2 * D:])
    h = jnp.tanh(h + b1_ref[:])
    out_ref[:] = _tdot(h, w2_ref[:]) + b2_ref[:]


def _mlp(self_feats, a0, a1, c0, c1, w1, b1, w2, b2):
    row_spec = pl.BlockSpec((BLK, D), lambda i: (i, 0))
    cnt_spec = pl.BlockSpec((BLK, CW), lambda i: (i, 0))

    def full(shape):
        return pl.BlockSpec(shape, lambda *_: (0,) * len(shape))

    return pl.pallas_call(
        _mlp_body,
        grid=(B // BLK,),
        in_specs=[row_spec, row_spec, row_spec, cnt_spec, cnt_spec,
                  full((D, 3 * D)), full((D,)), full((D, D)), full((D,))],
        out_specs=row_spec,
        out_shape=jax.ShapeDtypeStruct((B, D), jnp.float32),
    )(self_feats, a0, a1, c0, c1, w1, b1, w2, b2)


def kernel(nodes, edge_index_0, edge_index_1, feat_table, W1, b1, W2, b2):
    nodes_r = nodes.astype(jnp.int32).reshape(SELF_TILES, NSCH, SCH)
    e0 = edge_index_0.astype(jnp.int32).reshape(2, NS, NCH, CH)
    e1 = edge_index_1.astype(jnp.int32).reshape(2, NS, NCH, CH)
    featl = feat_table[:, :HD]
    featr = feat_table[:, HD:]

    self_rows, acc, cnt = _aggregate(
        feat_table, featl, featr, nodes_r, e0, e1,
        jnp.ones((CH, CW), jnp.float32))

    return _mlp(self_rows, acc[0], acc[1], cnt[0], cnt[1], W1, b1, W2, b2)


# speedup vs baseline: 1.1172x; 1.0725x over previous
import functools

import jax
import jax.numpy as jnp
from jax import lax
from jax.experimental import pallas as pl
from jax.experimental.pallas import tpu as pltpu
from jax.experimental.pallas import tpu_sc as plsc

B = 10000
D = 128
HD = D // 2
CW = 8
E = 160000
NREL = 2
NC = 2
NS = 16
NW = NC * NS
LANES = 16

CH = 125
EDGES_PER_TILE = E // NS
NCH = EDGES_PER_TILE // CH

B_PAD = 10240
SEG_ROWS = B_PAD // NS
ZCH = 128
NZ = SEG_ROWS // ZCH
SELF_TILES = 25
SELF_PER_TILE = B // SELF_TILES
SCH = 80
NSCH = SELF_PER_TILE // SCH

_mesh = plsc.VectorSubcoreMesh(core_axis_name="c", subcore_axis_name="s")


@functools.partial(
    pl.kernel,
    out_type=(
        jax.ShapeDtypeStruct((B, D), jnp.float32),
        jax.ShapeDtypeStruct((NREL, B_PAD, D), jnp.float32),
        jax.ShapeDtypeStruct((NREL, B_PAD, CW), jnp.float32),
    ),
    mesh=_mesh,
    compiler_params=pltpu.CompilerParams(use_tc_tiling_on_sc=False),
    scratch_types=[
        pltpu.VMEM((NCH, CH), jnp.int32),
        pltpu.VMEM((NCH, CH), jnp.int32),
        pltpu.VMEM((ZCH, HD), jnp.float32),
        pltpu.VMEM((ZCH, HD), jnp.float32),
        pltpu.VMEM((ZCH, HD), jnp.float32),
        pltpu.VMEM((ZCH, HD), jnp.float32),
        pltpu.VMEM((ZCH, HD), jnp.float32),
        pltpu.VMEM((ZCH, HD), jnp.float32),
        pltpu.VMEM((CH, CW), jnp.float32),
        pltpu.VMEM((NSCH, SCH), jnp.int32),
        pltpu.VMEM((SCH, D), jnp.float32),
        pltpu.VMEM((ZCH, LANES), jnp.float32),
        pltpu.VMEM_SHARED((B_PAD, HD), jnp.float32),
        pltpu.VMEM_SHARED((B_PAD, CW), jnp.float32),
        pltpu.SemaphoreType.DMA,
        pltpu.SemaphoreType.DMA,
        pltpu.SemaphoreType.DMA,
    ],
)
def _aggregate(feat_hbm, featl_hbm, featr_hbm, nodes_hbm, e0_hbm, e1_hbm,
               ones_hbm,
               self_out, acc_out, cnt_out,
               src_v, dst_v, rows_a, rows_b, rows_e, rows_f, rows_c, rows_d,
               ones_v, nidx_v, srows_v,
               zcnt_v, acc_sh, cnt_sh, sem_g, sem_s, sem_c):
    c = lax.axis_index("c")
    s = lax.axis_index("s")
    wid = c * NS + s

    zeros16 = jnp.zeros((LANES,), jnp.float32)
    ones16 = jnp.ones((LANES,), jnp.float32)

    def _zero_rows_a():
        def _zr(i, carry):
            for j in range(HD // LANES):
                rows_a[i, pl.ds(j * LANES, LANES)] = zeros16
            return carry

        lax.fori_loop(0, ZCH, _zr, 0)

    _zero_rows_a()

    def _init_row(i, carry):
        zcnt_v[i, :] = zeros16
        return carry

    lax.fori_loop(0, ZCH, _init_row, 0)

    pltpu.sync_copy(ones_hbm, ones_v)

    def _zero_acc():
        for j in range(NZ):
            pltpu.sync_copy(
                rows_a, acc_sh.at[pl.ds(s * SEG_ROWS + j * ZCH, ZCH)])

    _zero_acc()
    for j in range(NZ):
        pltpu.sync_copy(zcnt_v.at[pl.ds(0, ZCH), pl.ds(0, CW)],
                        cnt_sh.at[pl.ds(s * SEG_ROWS + j * ZCH, ZCH)])
    plsc.subcore_barrier()

    @pl.when(c == 0)
    def _():
        pltpu.sync_copy(e0_hbm.at[1, s], src_v)
        pltpu.sync_copy(e0_hbm.at[0, s], dst_v)

    @pl.when(c == 1)
    def _():
        pltpu.sync_copy(e1_hbm.at[1, s], src_v)
        pltpu.sync_copy(e1_hbm.at[0, s], dst_v)

    bufs = (rows_a, rows_b, rows_e, rows_f, rows_c, rows_d)
    NB = 6
    DEPTH = 3
    ones_pay = ones_v

    def _edge_phase(table_hbm, cnt_lo, cnt_hi):
        def _gather_src(j):
            return table_hbm.at[src_v.at[j]]

        def _rows(buf):
            return buf.at[pl.ds(0, CH)]

        for t in range(DEPTH):
            pltpu.async_copy(_gather_src(t), _rows(bufs[t]), sem_g)

        def _hex(i, carry):
            for t in range(NB):
                j = NB * i + t
                buf = bufs[t]
                bufd = bufs[(t + DEPTH) % NB]
                pltpu.make_async_copy(
                    _gather_src(j), _rows(buf), sem_g).wait()
                pltpu.async_copy(
                    _rows(buf), acc_sh.at[dst_v.at[j]], sem_s, add=True)

                @pl.when(jnp.logical_and(cnt_lo <= j, j < cnt_hi))
                def _():
                    pltpu.async_copy(
                        ones_pay, cnt_sh.at[dst_v.at[j]], sem_c, add=True)

                @pl.when(j >= DEPTH)
                def _():
                    pltpu.make_async_copy(
                        _rows(bufd), acc_sh.at[dst_v.at[j - DEPTH]],
                        sem_s).wait()

                @pl.when(j + DEPTH < NCH)
                def _():
                    pltpu.async_copy(
                        _gather_src(j + DEPTH), _rows(bufd), sem_g)
            return carry

        lax.fori_loop(0, NCH // NB, _hex, 0)

        for j in range(NCH - NCH % NB, NCH):
            buf = bufs[j % NB]
            bufd = bufs[(j + DEPTH) % NB]
            pltpu.make_async_copy(_gather_src(j), _rows(buf), sem_g).wait()
            pltpu.async_copy(
                _rows(buf), acc_sh.at[dst_v.at[j]], sem_s, add=True)

            @pl.when(jnp.logical_and(cnt_lo <= j, j < cnt_hi))
            def _():
                pltpu.async_copy(
                    ones_pay, cnt_sh.at[dst_v.at[j]], sem_c, add=True)

            pltpu.make_async_copy(
                _rows(bufd), acc_sh.at[dst_v.at[j - DEPTH]], sem_s).wait()
        for j in range(NCH - DEPTH, NCH):
            pltpu.make_async_copy(
                _rows(bufs[j % NB]), acc_sh.at[dst_v.at[j]], sem_s).wait()

        def _drain(j, carry):
            @pl.when(jnp.logical_and(cnt_lo <= j, j < cnt_hi))
            def _():
                pltpu.make_async_copy(
                    ones_pay, cnt_sh.at[dst_v.at[j]], sem_c).wait()
            return carry

        lax.fori_loop(0, NCH, _drain, 0)

    _edge_phase(featl_hbm, 0, NCH // 2)

    plsc.subcore_barrier()
    for j in range(NZ):
        base = s * SEG_ROWS + j * ZCH
        pltpu.sync_copy(acc_sh.at[pl.ds(base, ZCH)],
                        acc_out.at[c, pl.ds(base, ZCH), pl.ds(0, HD)])
    _zero_rows_a()
    _zero_acc()
    plsc.subcore_barrier()

    _edge_phase(featr_hbm, NCH // 2, NCH)

    @pl.when(wid < SELF_TILES)
    def _():
        pltpu.sync_copy(nodes_hbm.at[wid], nidx_v)
        for t in range(NSCH):
            pltpu.async_copy(feat_hbm.at[nidx_v.at[t]], srows_v, sem_g).wait()
            pltpu.sync_copy(
                srows_v,
                self_out.at[pl.ds(wid * SELF_PER_TILE + t * SCH, SCH)])

    plsc.subcore_barrier()
    for j in range(NZ):
        base = s * SEG_ROWS + j * ZCH
        pltpu.sync_copy(acc_sh.at[pl.ds(base, ZCH)],
                        acc_out.at[c, pl.ds(base, ZCH), pl.ds(HD, HD)])
        pltpu.sync_copy(cnt_sh.at[pl.ds(base, ZCH)],
                        cnt_out.at[c, pl.ds(base, ZCH)])


BLK = 2000


def _tdot(x, w):
    return lax.dot_general(x, w, (((1,), (1,)), ((), ())),
                           preferred_element_type=jnp.float32)


def _mlp_body(self_ref, a0_ref, a1_ref, c0_ref, c1_ref,
              w1_ref, b1_ref, w2_ref, b2_ref, out_ref):
    inv0 = 1.0 / jnp.maximum(c0_ref[0][:, 0:1], 1.0)
    inv1 = 1.0 / jnp.maximum(c1_ref[0][:, 0:1], 1.0)
    h = _tdot(self_ref[:], w1_ref[:, :D])
    h = h + _tdot(a0_ref[0] * inv0, w1_ref[:, D:2 * D])
    h = h + _tdot(a1_ref[0] * inv1, w1_ref[:, 2 * D:])
    h = jnp.tanh(h + b1_ref[:])
    out_ref[:] = _tdot(h, w2_ref[:]) + b2_ref[:]


def _mlp(self_feats, acc3, cnt3, w1, b1, w2, b2):
    row_spec = pl.BlockSpec((BLK, D), lambda i: (i, 0))

    def rel(k, width):
        return pl.BlockSpec((1, BLK, width), lambda i, _k=k: (_k, i, 0))

    def full(shape):
        return pl.BlockSpec(shape, lambda *_: (0,) * len(shape))

    return pl.pallas_call(
        _mlp_body,
        grid=(B // BLK,),
        in_specs=[row_spec, rel(0, D), rel(1, D), rel(0, CW), rel(1, CW),
                  full((D, 3 * D)), full((D,)), full((D, D)), full((D,))],
        out_specs=row_spec,
        out_shape=jax.ShapeDtypeStruct((B, D), jnp.float32),
    )(self_feats, acc3, acc3, cnt3, cnt3, w1, b1, w2, b2)


def kernel(nodes, edge_index_0, edge_index_1, feat_table, W1, b1, W2, b2):
    nodes_r = nodes.astype(jnp.int32).reshape(SELF_TILES, NSCH, SCH)
    e0 = edge_index_0.astype(jnp.int32).reshape(2, NS, NCH, CH)
    e1 = edge_index_1.astype(jnp.int32).reshape(2, NS, NCH, CH)
    featl = feat_table[:, :HD]
    featr = feat_table[:, HD:]

    self_rows, acc, cnt = _aggregate(
        feat_table, featl, featr, nodes_r, e0, e1,
        jnp.ones((CH, CW), jnp.float32))

    return _mlp(self_rows, acc, cnt, W1, b1, W2, b2)


# speedup vs baseline: 1.1180x; 1.0007x over previous
import functools

import jax
import jax.numpy as jnp
from jax import lax
from jax.experimental import pallas as pl
from jax.experimental.pallas import tpu as pltpu
from jax.experimental.pallas import tpu_sc as plsc

B = 10000
D = 128
HD = D // 2
CW = 8
E = 160000
NREL = 2
NC = 2
NS = 16
NW = NC * NS
LANES = 16

CH = 125
EDGES_PER_TILE = E // NS
NCH = EDGES_PER_TILE // CH

B_PAD = 10240
SEG_ROWS = B_PAD // NS
ZCH = 128
NZ = SEG_ROWS // ZCH
SELF_TILES = 25
SELF_PER_TILE = B // SELF_TILES
SCH = 80
NSCH = SELF_PER_TILE // SCH

_mesh = plsc.VectorSubcoreMesh(core_axis_name="c", subcore_axis_name="s")


@functools.partial(
    pl.kernel,
    out_type=(
        jax.ShapeDtypeStruct((B, D), jnp.float32),
        jax.ShapeDtypeStruct((NREL, B_PAD, D), jnp.float32),
        jax.ShapeDtypeStruct((NREL, B_PAD, CW), jnp.float32),
    ),
    mesh=_mesh,
    compiler_params=pltpu.CompilerParams(use_tc_tiling_on_sc=False),
    scratch_types=[
        pltpu.VMEM((NCH, CH), jnp.int32),
        pltpu.VMEM((NCH, CH), jnp.int32),
        pltpu.VMEM((ZCH, HD), jnp.float32),
        pltpu.VMEM((ZCH, HD), jnp.float32),
        pltpu.VMEM((ZCH, HD), jnp.float32),
        pltpu.VMEM((ZCH, HD), jnp.float32),
        pltpu.VMEM((ZCH, HD), jnp.float32),
        pltpu.VMEM((ZCH, HD), jnp.float32),
        pltpu.VMEM((CH, CW), jnp.float32),
        pltpu.VMEM((NSCH, SCH), jnp.int32),
        pltpu.VMEM((SCH, D), jnp.float32),
        pltpu.VMEM((ZCH, LANES), jnp.float32),
        pltpu.VMEM_SHARED((B_PAD, HD), jnp.float32),
        pltpu.VMEM_SHARED((B_PAD, CW), jnp.float32),
        pltpu.SemaphoreType.DMA,
        pltpu.SemaphoreType.DMA,
        pltpu.SemaphoreType.DMA,
    ],
)
def _aggregate(feat_hbm, featl_hbm, featr_hbm, nodes_hbm, e0_hbm, e1_hbm,
               ones_hbm,
               self_out, acc_out, cnt_out,
               src_v, dst_v, rows_a, rows_b, rows_e, rows_f, rows_c, rows_d,
               ones_v, nidx_v, srows_v,
               zcnt_v, acc_sh, cnt_sh, sem_g, sem_s, sem_c):
    c = lax.axis_index("c")
    s = lax.axis_index("s")
    wid = c * NS + s

    zeros16 = jnp.zeros((LANES,), jnp.float32)

    def _zero_rows_a():
        def _zr(i, carry):
            for j in range(HD // LANES):
                rows_a[i, pl.ds(j * LANES, LANES)] = zeros16
            return carry

        lax.fori_loop(0, ZCH, _zr, 0)

    _zero_rows_a()

    def _init_row(i, carry):
        zcnt_v[i, :] = zeros16
        return carry

    lax.fori_loop(0, ZCH, _init_row, 0)

    pltpu.sync_copy(ones_hbm, ones_v)

    def _zero_acc():
        for j in range(NZ):
            pltpu.sync_copy(
                rows_a, acc_sh.at[pl.ds(s * SEG_ROWS + j * ZCH, ZCH)])

    _zero_acc()
    for j in range(NZ):
        pltpu.sync_copy(zcnt_v.at[pl.ds(0, ZCH), pl.ds(0, CW)],
                        cnt_sh.at[pl.ds(s * SEG_ROWS + j * ZCH, ZCH)])
    plsc.subcore_barrier()

    @pl.when(c == 0)
    def _():
        pltpu.sync_copy(e0_hbm.at[1, s], src_v)
        pltpu.sync_copy(e0_hbm.at[0, s], dst_v)

    @pl.when(c == 1)
    def _():
        pltpu.sync_copy(e1_hbm.at[1, s], src_v)
        pltpu.sync_copy(e1_hbm.at[0, s], dst_v)

    bufs = (rows_a, rows_b, rows_e, rows_f, rows_c, rows_d)
    NB = 6
    DEPTH = 3
    ones_pay = ones_v

    def _edge_phase(table_hbm, cnt_lo, cnt_hi):
        def _gather_src(j):
            return table_hbm.at[src_v.at[j]]

        def _rows(buf):
            return buf.at[pl.ds(0, CH)]

        for t in range(DEPTH):
            pltpu.async_copy(_gather_src(t), _rows(bufs[t]), sem_g)

        def _hex(i, carry):
            for t in range(NB):
                j = NB * i + t
                buf = bufs[t]
                bufd = bufs[(t + DEPTH) % NB]
                pltpu.make_async_copy(
                    _gather_src(j), _rows(buf), sem_g).wait()
                pltpu.async_copy(
                    _rows(buf), acc_sh.at[dst_v.at[j]], sem_s, add=True)

                @pl.when(jnp.logical_and(cnt_lo <= j, j < cnt_hi))
                def _():
                    pltpu.async_copy(
                        ones_pay, cnt_sh.at[dst_v.at[j]], sem_c, add=True)

                @pl.when(j >= DEPTH)
                def _():
                    pltpu.make_async_copy(
                        _rows(bufd), acc_sh.at[dst_v.at[j - DEPTH]],
                        sem_s).wait()

                @pl.when(j + DEPTH < NCH)
                def _():
                    pltpu.async_copy(
                        _gather_src(j + DEPTH), _rows(bufd), sem_g)
            return carry

        lax.fori_loop(0, NCH // NB, _hex, 0)

        for j in range(NCH - NCH % NB, NCH):
            buf = bufs[j % NB]
            bufd = bufs[(j + DEPTH) % NB]
            pltpu.make_async_copy(_gather_src(j), _rows(buf), sem_g).wait()
            pltpu.async_copy(
                _rows(buf), acc_sh.at[dst_v.at[j]], sem_s, add=True)

            @pl.when(jnp.logical_and(cnt_lo <= j, j < cnt_hi))
            def _():
                pltpu.async_copy(
                    ones_pay, cnt_sh.at[dst_v.at[j]], sem_c, add=True)

            pltpu.make_async_copy(
                _rows(bufd), acc_sh.at[dst_v.at[j - DEPTH]], sem_s).wait()
        for j in range(NCH - DEPTH, NCH):
            pltpu.make_async_copy(
                _rows(bufs[j % NB]), acc_sh.at[dst_v.at[j]], sem_s).wait()

        def _drain(j, carry):
            @pl.when(jnp.logical_and(cnt_lo <= j, j < cnt_hi))
            def _():
                pltpu.make_async_copy(
                    ones_pay, cnt_sh.at[dst_v.at[j]], sem_c).wait()
            return carry

        lax.fori_loop(0, NCH, _drain, 0)

    _edge_phase(featl_hbm, 0, NCH // 2)

    plsc.subcore_barrier()
    for j in range(NZ):
        base = s * SEG_ROWS + j * ZCH
        pltpu.sync_copy(acc_sh.at[pl.ds(base, ZCH)],
                        acc_out.at[c, pl.ds(base, ZCH), pl.ds(0, HD)])
    _zero_rows_a()
    _zero_acc()
    plsc.subcore_barrier()

    _edge_phase(featr_hbm, NCH // 2, NCH)

    @pl.when(wid < SELF_TILES)
    def _():
        pltpu.sync_copy(nodes_hbm.at[wid], nidx_v)
        for t in range(NSCH):
            pltpu.async_copy(feat_hbm.at[nidx_v.at[t]], srows_v, sem_g).wait()
            pltpu.sync_copy(
                srows_v,
                self_out.at[pl.ds(wid * SELF_PER_TILE + t * SCH, SCH)])

    plsc.subcore_barrier()
    for j in range(NZ):
        base = s * SEG_ROWS + j * ZCH
        pltpu.sync_copy(acc_sh.at[pl.ds(base, ZCH)],
                        acc_out.at[c, pl.ds(base, ZCH), pl.ds(HD, HD)])
        pltpu.sync_copy(cnt_sh.at[pl.ds(base, ZCH)],
                        cnt_out.at[c, pl.ds(base, ZCH)])


BLK = 2000


def _tdot(x, w):
    return lax.dot_general(x, w, (((1,), (1,)), ((), ())),
                           preferred_element_type=jnp.float32)


def _mlp_body(self_ref, a0_ref, a1_ref, c0_ref, c1_ref,
              w1_ref, b1_ref, w2_ref, b2_ref, out_ref):
    inv0 = 1.0 / jnp.maximum(c0_ref[0][:, 0:1], 1.0)
    inv1 = 1.0 / jnp.maximum(c1_ref[0][:, 0:1], 1.0)
    h = _tdot(self_ref[:], w1_ref[:, :D])
    h = h + _tdot(a0_ref[0] * inv0, w1_ref[:, D:2 * D])
    h = h + _tdot(a1_ref[0] * inv1, w1_ref[:, 2 * D:])
    h = jnp.tanh(h + b1_ref[:])
    out_ref[:] = _tdot(h, w2_ref[:]) + b2_ref[:]


def _mlp(self_feats, acc3, cnt3, w1, b1, w2, b2):
    row_spec = pl.BlockSpec((BLK, D), lambda i: (i, 0))

    def rel(k, width):
        return pl.BlockSpec((1, BLK, width), lambda i, _k=k: (_k, i, 0))

    def full(shape):
        return pl.BlockSpec(shape, lambda *_: (0,) * len(shape))

    return pl.pallas_call(
        _mlp_body,
        grid=(B // BLK,),
        in_specs=[row_spec, rel(0, D), rel(1, D), rel(0, CW), rel(1, CW),
                  full((D, 3 * D)), full((D,)), full((D, D)), full((D,))],
        out_specs=row_spec,
        out_shape=jax.ShapeDtypeStruct((B, D), jnp.float32),
    )(self_feats, acc3, acc3, cnt3, cnt3, w1, b1, w2, b2)


def kernel(nodes, edge_index_0, edge_index_1, feat_table, W1, b1, W2, b2):
    nodes_r = nodes.astype(jnp.int32).reshape(SELF_TILES, NSCH, SCH)
    e0 = edge_index_0.astype(jnp.int32).reshape(2, NS, NCH, CH)
    e1 = edge_index_1.astype(jnp.int32).reshape(2, NS, NCH, CH)
    featl = feat_table[:, :HD]
    featr = feat_table[:, HD:]

    self_rows, acc, cnt = _aggregate(
        feat_table, featl, featr, nodes_r, e0, e1,
        jnp.ones((CH, CW), jnp.float32))

    return _mlp(self_rows, acc, cnt, W1, b1, W2, b2)
